# Initial kernel scaffold; baseline (speedup 1.0000x reference)
import jax
import jax.numpy as jnp
from jax.experimental import pallas as pl


def kernel(pred_dsmat, pred_perm, pred_perm_att, gt_perm, src_ns, tgt_ns):
    raise NotImplementedError("write your pallas kernel here")


# speedup vs baseline: 1.8716x; 1.8716x over previous
import functools

import jax
import jax.numpy as jnp
from jax import lax
from jax.experimental import pallas as pl
from jax.experimental.pallas import tpu as pltpu
from jax.experimental.pallas import tpu_sc as plsc

_B = 4
_N = 2048
_REG_RATIO = 0.1
_RB = 256
_NR = _N // _RB
_NW = 32
_CHUNK = _B * _N // _NW
_GROUPS = _CHUNK // 16


def _tc_body(src_ref, tgt_ref, pred_ref, gt_ref, att_ref,
             cg_ref, ca_ref, cgt_ref, p1_ref, s1_ref):
    b = pl.program_id(0)
    ir = pl.program_id(1)
    s = src_ref[b]
    t = tgt_ref[b]
    P = pred_ref[0]
    G = gt_ref[0]
    A = att_ref[0]
    rows = lax.broadcasted_iota(jnp.int32, (_RB, _N), 0) + ir * _RB
    cols = lax.broadcasted_iota(jnp.int32, (_RB, _N), 1)
    rv = rows < s
    region = rv & (cols < t)
    l1mp = jnp.maximum(jnp.log(1.0 - P), -100.0)
    s1 = jnp.sum(jnp.where(region, -l1mp, 0.0))
    colf = cols.astype(jnp.float32)
    rowf = rows.astype(jnp.float32)
    cg = jnp.sum(G * colf, axis=1)
    ca = jnp.sum(A * colf, axis=1)
    p1 = jnp.sum(P * G, axis=1)
    lp1 = jnp.maximum(jnp.log(p1), -100.0)
    l1mp1 = jnp.maximum(jnp.log(1.0 - p1), -100.0)
    corr = jnp.sum(jnp.where(rv[:, 0], l1mp1 - lp1, 0.0))
    cg_ref[0, 0, :] = cg.astype(jnp.int32)
    ca_ref[0, 0, :] = ca.astype(jnp.int32)
    p1_ref[0, 0, :] = p1
    cgt_part = jnp.sum(G * rowf, axis=0)

    @pl.when(ir == 0)
    def _init():
        cgt_ref[0, 0, :] = cgt_part

    @pl.when(ir != 0)
    def _acc():
        cgt_ref[0, 0, :] = cgt_ref[0, 0, :] + cgt_part

    lane = lax.broadcasted_iota(jnp.int32, (1, 128), 1)
    s1_ref[0, :, :] = jnp.where(lane == 0, s1 + corr, 0.0)


def _tc_pass(pred, gt, att, src_i, tgt_i):
    blk = lambda b, ir: (b, ir, 0)
    out = pl.pallas_call(
        _tc_body,
        grid=(_B, _NR),
        in_specs=[
            pl.BlockSpec(memory_space=pltpu.SMEM),
            pl.BlockSpec(memory_space=pltpu.SMEM),
            pl.BlockSpec((1, _RB, _N), blk),
            pl.BlockSpec((1, _RB, _N), blk),
            pl.BlockSpec((1, _RB, _N), blk),
        ],
        out_specs=[
            pl.BlockSpec((1, 1, _RB), lambda b, ir: (b * _NR + ir, 0, 0)),
            pl.BlockSpec((1, 1, _RB), lambda b, ir: (b * _NR + ir, 0, 0)),
            pl.BlockSpec((1, 1, _N), lambda b, ir: (b, 0, 0)),
            pl.BlockSpec((1, 1, _RB), lambda b, ir: (b * _NR + ir, 0, 0)),
            pl.BlockSpec((1, 1, 128), lambda b, ir: (b * _NR + ir, 0, 0)),
        ],
        out_shape=[
            jax.ShapeDtypeStruct((_B * _NR, 1, _RB), jnp.int32),
            jax.ShapeDtypeStruct((_B * _NR, 1, _RB), jnp.int32),
            jax.ShapeDtypeStruct((_B, 1, _N), jnp.float32),
            jax.ShapeDtypeStruct((_B * _NR, 1, _RB), jnp.float32),
            jax.ShapeDtypeStruct((_B * _NR, 1, 128), jnp.float32),
        ],
        compiler_params=pltpu.CompilerParams(
            dimension_semantics=("arbitrary", "arbitrary")),
    )(src_i, tgt_i, pred, gt, att)
    return out


def _sc_body(pred2d, cg_hbm, ca_hbm, cgt_hbm, p1_hbm, src_hbm, out_hbm,
             ca_v, p1_v, cgt_v, cg_v, src_v, rows_v, acc_v, sem):
    wid = lax.axis_index("s") * 2 + lax.axis_index("c")
    base = wid * _CHUNK
    b = base // _N
    ibase = base - b * _N
    pltpu.sync_copy(ca_hbm.at[pl.ds(base, _CHUNK)], ca_v)
    pltpu.sync_copy(p1_hbm.at[pl.ds(base, _CHUNK)], p1_v)
    pltpu.sync_copy(cgt_hbm.at[pl.ds(b * _N, _N)], cgt_v)
    pltpu.sync_copy(cg_hbm.at[pl.ds(b * _N, _N)], cg_v)
    pltpu.sync_copy(src_hbm, src_v)
    s_vec = plsc.load_gather(src_v, [jnp.full((16,), b, jnp.int32)])
    lanes = lax.iota(jnp.int32, 16)
    rowbase = b * ((_N * _N) // 128)
    acc = jnp.zeros((16,), jnp.float32)
    for g in range(_GROUPS):
        i16 = ibase + g * 16 + lanes
        a16 = ca_v[pl.ds(g * 16, 16)]
        m16 = plsc.load_gather(cgt_v, [a16]).astype(jnp.int32)
        sc16 = plsc.load_gather(cg_v, [m16])
        ridx = rowbase + i16 * (_N // 128) + lax.shift_right_logical(sc16, 7)
        pltpu.async_copy(pred2d.at[ridx], rows_v, sem).wait()
        col16 = lax.bitwise_and(sc16, 127)
        p2 = plsc.load_gather(rows_v, [lanes, col16])
        p1g = p1_v[pl.ds(g * 16, 16)]
        mask = (i16 < s_vec) & (m16 != i16)
        acc = acc + jnp.where(mask, p1g - p2, jnp.zeros((16,), jnp.float32))
    acc_v[...] = acc
    pltpu.sync_copy(acc_v, out_hbm.at[wid])


def _make_sc_kernel():
    return functools.partial(
        pl.kernel,
        mesh=plsc.VectorSubcoreMesh(core_axis_name="c", subcore_axis_name="s"),
        out_type=jax.ShapeDtypeStruct((_NW, 16), jnp.float32),
        compiler_params=pltpu.CompilerParams(needs_layout_passes=False),
        scratch_types=[
            pltpu.VMEM((_CHUNK,), jnp.int32),
            pltpu.VMEM((_CHUNK,), jnp.float32),
            pltpu.VMEM((_N,), jnp.float32),
            pltpu.VMEM((_N,), jnp.int32),
            pltpu.VMEM((16,), jnp.int32),
            pltpu.VMEM((16, 128), jnp.float32),
            pltpu.VMEM((16,), jnp.float32),
            pltpu.SemaphoreType.DMA,
        ],
    )(_sc_body)


def kernel(pred_dsmat, pred_perm, pred_perm_att, gt_perm, src_ns, tgt_ns):
    pred = pred_dsmat.astype(jnp.float32)
    gt = gt_perm.astype(jnp.float32)
    att = pred_perm_att.astype(jnp.float32)
    src_i = src_ns.astype(jnp.int32)
    tgt_i = tgt_ns.astype(jnp.int32)
    cg, ca, cgt, p1, s1 = _tc_pass(pred, gt, att, src_i, tgt_i)
    pred2d = pred.reshape(_B * _N * _N // 128, 128)
    src_pad = jnp.zeros((16,), jnp.int32).at[:_B].set(src_i)
    reg_parts = _make_sc_kernel()(
        pred2d,
        cg.reshape(_B * _N),
        ca.reshape(_B * _N),
        cgt.reshape(_B * _N),
        p1.reshape(_B * _N),
        src_pad,
    )
    total = jnp.sum(s1) - _REG_RATIO * jnp.sum(reg_parts)
    nsum = jnp.sum(src_i.astype(jnp.float32))
    return total / nsum


# speedup vs baseline: 2.0036x; 1.0706x over previous
import functools

import jax
import jax.numpy as jnp
from jax import lax
from jax.experimental import pallas as pl
from jax.experimental.pallas import tpu as pltpu
from jax.experimental.pallas import tpu_sc as plsc

_B = 4
_N = 2048
_REG_RATIO = 0.1
_RB = 256
_NR = _N // _RB
_NW = 32
_CHUNK = _B * _N // _NW
_GROUPS = _CHUNK // 16


def _tc_body(src_ref, tgt_ref, pred_ref, gt_ref, att_ref,
             cg_ref, ca_ref, cgt_ref, p1_ref, s1_ref, s1_acc):
    b = pl.program_id(0)
    ir = pl.program_id(1)
    s = src_ref[b]
    t = tgt_ref[b]
    P = pred_ref[0]
    G = gt_ref[0]
    A = att_ref[0]
    rows = lax.broadcasted_iota(jnp.int32, (_RB, _N), 0) + ir * _RB
    cols = lax.broadcasted_iota(jnp.int32, (_RB, _N), 1)
    rv = rows < s
    region = rv & (cols < t)
    l1mp = jnp.maximum(jnp.log(1.0 - P), -100.0)
    s1 = jnp.sum(jnp.where(region, -l1mp, 0.0))
    colf = cols.astype(jnp.float32)
    rowf = rows.astype(jnp.float32)
    cg = jnp.sum(G * colf, axis=1)
    ca = jnp.sum(A * colf, axis=1)
    p1 = jnp.sum(P * G, axis=1)
    lp1 = jnp.maximum(jnp.log(p1), -100.0)
    l1mp1 = jnp.maximum(jnp.log(1.0 - p1), -100.0)
    corr = jnp.sum(jnp.where(rv[:, 0], l1mp1 - lp1, 0.0))
    cg_ref[0, 0, :] = cg.astype(jnp.int32)
    ca_ref[0, 0, :] = ca.astype(jnp.int32)
    p1_ref[0, 0, :] = p1
    cgt_part = jnp.sum(G * rowf, axis=0)

    @pl.when(ir == 0)
    def _init():
        cgt_ref[0, 0, :] = cgt_part

    @pl.when(ir != 0)
    def _acc():
        cgt_ref[0, 0, :] = cgt_ref[0, 0, :] + cgt_part

    part = s1 + corr

    @pl.when((b == 0) & (ir == 0))
    def _first():
        s1_acc[0] = part

    @pl.when((b != 0) | (ir != 0))
    def _rest():
        s1_acc[0] = s1_acc[0] + part

    @pl.when((b == _B - 1) & (ir == _NR - 1))
    def _flush():
        s1_ref[0] = s1_acc[0]


def _tc_pass(pred, gt, att, src_i, tgt_i):
    blk = lambda b, ir: (b, ir, 0)
    out = pl.pallas_call(
        _tc_body,
        grid=(_B, _NR),
        in_specs=[
            pl.BlockSpec(memory_space=pltpu.SMEM),
            pl.BlockSpec(memory_space=pltpu.SMEM),
            pl.BlockSpec((1, _RB, _N), blk),
            pl.BlockSpec((1, _RB, _N), blk),
            pl.BlockSpec((1, _RB, _N), blk),
        ],
        out_specs=[
            pl.BlockSpec((1, 1, _RB), lambda b, ir: (b * _NR + ir, 0, 0)),
            pl.BlockSpec((1, 1, _RB), lambda b, ir: (b * _NR + ir, 0, 0)),
            pl.BlockSpec((1, 1, _N), lambda b, ir: (b, 0, 0)),
            pl.BlockSpec((1, 1, _RB), lambda b, ir: (b * _NR + ir, 0, 0)),
            pl.BlockSpec(memory_space=pltpu.SMEM),
        ],
        out_shape=[
            jax.ShapeDtypeStruct((_B * _NR, 1, _RB), jnp.int32),
            jax.ShapeDtypeStruct((_B * _NR, 1, _RB), jnp.int32),
            jax.ShapeDtypeStruct((_B, 1, _N), jnp.float32),
            jax.ShapeDtypeStruct((_B * _NR, 1, _RB), jnp.float32),
            jax.ShapeDtypeStruct((1,), jnp.float32),
        ],
        scratch_shapes=[pltpu.SMEM((1,), jnp.float32)],
        compiler_params=pltpu.CompilerParams(
            dimension_semantics=("arbitrary", "arbitrary")),
    )(src_i, tgt_i, pred, gt, att)
    return out


def _sc_body(pred2d, cg_hbm, ca_hbm, cgt_hbm, p1_hbm, src_hbm, out_hbm,
             ca_v, p1_v, cgt_v, cg_v, src_v, idx_v, col_v, msk_v,
             rows0_v, rows1_v, acc_v, sem):
    wid = lax.axis_index("s") * 2 + lax.axis_index("c")
    base = wid * _CHUNK
    b = base // _N
    ibase = base - b * _N
    pltpu.sync_copy(ca_hbm.at[pl.ds(base, _CHUNK)], ca_v)
    pltpu.sync_copy(p1_hbm.at[pl.ds(base, _CHUNK)], p1_v)
    pltpu.sync_copy(cgt_hbm.at[pl.ds(b * _N, _N)], cgt_v)
    pltpu.sync_copy(cg_hbm.at[pl.ds(b * _N, _N)], cg_v)
    pltpu.sync_copy(src_hbm, src_v)
    s_vec = plsc.load_gather(src_v, [jnp.full((16,), b, jnp.int32)])
    lanes = lax.iota(jnp.int32, 16)
    rowbase = b * ((_N * _N) // 128)
    for g in range(_GROUPS):
        i16 = ibase + g * 16 + lanes
        a16 = ca_v[pl.ds(g * 16, 16)]
        m16 = plsc.load_gather(cgt_v, [a16]).astype(jnp.int32)
        sc16 = plsc.load_gather(cg_v, [m16])
        ridx = rowbase + i16 * (_N // 128) + lax.shift_right_logical(sc16, 7)
        idx_v[g // 8, pl.ds((g % 8) * 16, 16)] = ridx
        col_v[pl.ds(g * 16, 16)] = lax.bitwise_and(sc16, 127)
        mask = (i16 < s_vec) & (m16 != i16)
        msk_v[pl.ds(g * 16, 16)] = mask.astype(jnp.int32)
    d0 = pltpu.async_copy(pred2d.at[idx_v.at[0]], rows0_v, sem)
    d1 = pltpu.async_copy(pred2d.at[idx_v.at[1]], rows1_v, sem)
    d0.wait()
    d1.wait()
    acc = jnp.zeros((16,), jnp.float32)
    for g in range(_GROUPS):
        loc = (g % 8) * 16 + lanes
        col16 = col_v[pl.ds(g * 16, 16)]
        if g < 8:
            p2 = plsc.load_gather(rows0_v, [loc, col16])
        else:
            p2 = plsc.load_gather(rows1_v, [loc, col16])
        p1g = p1_v[pl.ds(g * 16, 16)]
        mask = msk_v[pl.ds(g * 16, 16)] != 0
        acc = acc + jnp.where(mask, p1g - p2, jnp.zeros((16,), jnp.float32))
    acc_v[...] = acc
    pltpu.sync_copy(acc_v, out_hbm.at[wid])


def _make_sc_kernel():
    return functools.partial(
        pl.kernel,
        mesh=plsc.VectorSubcoreMesh(core_axis_name="c", subcore_axis_name="s"),
        out_type=jax.ShapeDtypeStruct((_NW, 16), jnp.float32),
        compiler_params=pltpu.CompilerParams(needs_layout_passes=False),
        scratch_types=[
            pltpu.VMEM((_CHUNK,), jnp.int32),
            pltpu.VMEM((_CHUNK,), jnp.float32),
            pltpu.VMEM((_N,), jnp.float32),
            pltpu.VMEM((_N,), jnp.int32),
            pltpu.VMEM((16,), jnp.int32),
            pltpu.VMEM((2, 128), jnp.int32),
            pltpu.VMEM((_CHUNK,), jnp.int32),
            pltpu.VMEM((_CHUNK,), jnp.int32),
            pltpu.VMEM((128, 128), jnp.float32),
            pltpu.VMEM((128, 128), jnp.float32),
            pltpu.VMEM((16,), jnp.float32),
            pltpu.SemaphoreType.DMA,
        ],
    )(_sc_body)


def kernel(pred_dsmat, pred_perm, pred_perm_att, gt_perm, src_ns, tgt_ns):
    pred = pred_dsmat.astype(jnp.float32)
    gt = gt_perm.astype(jnp.float32)
    att = pred_perm_att.astype(jnp.float32)
    src_i = src_ns.astype(jnp.int32)
    tgt_i = tgt_ns.astype(jnp.int32)
    cg, ca, cgt, p1, s1 = _tc_pass(pred, gt, att, src_i, tgt_i)
    pred2d = pred.reshape(_B * _N * _N // 128, 128)
    src_pad = jnp.zeros((16,), jnp.int32).at[:_B].set(src_i)
    reg_parts = _make_sc_kernel()(
        pred2d,
        cg.reshape(_B * _N),
        ca.reshape(_B * _N),
        cgt.reshape(_B * _N),
        p1.reshape(_B * _N),
        src_pad,
    )
    total = s1[0] - _REG_RATIO * jnp.sum(reg_parts)
    nsum = jnp.sum(src_i.astype(jnp.float32))
    return total / nsum


# speedup vs baseline: 2.1135x; 1.0548x over previous
import functools

import jax
import jax.numpy as jnp
from jax import lax
from jax.experimental import pallas as pl
from jax.experimental.pallas import tpu as pltpu
from jax.experimental.pallas import tpu_sc as plsc

_B = 4
_N = 2048
_REG_RATIO = 0.1
_RB = 512
_NR = _N // _RB
_NW = 32
_CHUNK = _B * _N // _NW
_GROUPS = _CHUNK // 16


def _tc_body(src_ref, tgt_ref, pred_ref, gt_ref, att_ref,
             cg_ref, ca_ref, cgt_ref, p1_ref, s1_ref, s1_acc):
    b = pl.program_id(0)
    ir = pl.program_id(1)
    s = src_ref[b]
    t = tgt_ref[b]
    P = pred_ref[0]
    G = gt_ref[0]
    A = att_ref[0]
    rows = lax.broadcasted_iota(jnp.int32, (_RB, _N), 0) + ir * _RB
    cols = lax.broadcasted_iota(jnp.int32, (_RB, _N), 1)
    rv = rows < s
    region = rv & (cols < t)
    l1mp = jnp.maximum(jnp.log(1.0 - P), -100.0)
    s1 = jnp.sum(jnp.where(region, -l1mp, 0.0))
    colf = cols.astype(jnp.float32)
    rowf = rows.astype(jnp.float32)
    cg = jnp.sum(G * colf, axis=1)
    ca = jnp.sum(A * colf, axis=1)
    p1 = jnp.sum(P * G, axis=1)
    lp1 = jnp.maximum(jnp.log(p1), -100.0)
    l1mp1 = jnp.maximum(jnp.log(1.0 - p1), -100.0)
    corr = jnp.sum(jnp.where(rv[:, 0], l1mp1 - lp1, 0.0))
    cg_ref[0, 0, :] = cg.astype(jnp.int32)
    ca_ref[0, 0, :] = ca.astype(jnp.int32)
    p1_ref[0, 0, :] = p1
    cgt_part = jnp.sum(G * rowf, axis=0)

    @pl.when(ir == 0)
    def _init():
        cgt_ref[0, 0, :] = cgt_part

    @pl.when(ir != 0)
    def _acc():
        cgt_ref[0, 0, :] = cgt_ref[0, 0, :] + cgt_part

    part = s1 + corr

    @pl.when((b == 0) & (ir == 0))
    def _first():
        s1_acc[0] = part

    @pl.when((b != 0) | (ir != 0))
    def _rest():
        s1_acc[0] = s1_acc[0] + part

    @pl.when((b == _B - 1) & (ir == _NR - 1))
    def _flush():
        s1_ref[0] = s1_acc[0]


def _tc_pass(pred, gt, att, src_i, tgt_i):
    blk = lambda b, ir: (b, ir, 0)
    out = pl.pallas_call(
        _tc_body,
        grid=(_B, _NR),
        in_specs=[
            pl.BlockSpec(memory_space=pltpu.SMEM),
            pl.BlockSpec(memory_space=pltpu.SMEM),
            pl.BlockSpec((1, _RB, _N), blk),
            pl.BlockSpec((1, _RB, _N), blk),
            pl.BlockSpec((1, _RB, _N), blk),
        ],
        out_specs=[
            pl.BlockSpec((1, 1, _RB), lambda b, ir: (b * _NR + ir, 0, 0)),
            pl.BlockSpec((1, 1, _RB), lambda b, ir: (b * _NR + ir, 0, 0)),
            pl.BlockSpec((1, 1, _N), lambda b, ir: (b, 0, 0)),
            pl.BlockSpec((1, 1, _RB), lambda b, ir: (b * _NR + ir, 0, 0)),
            pl.BlockSpec(memory_space=pltpu.SMEM),
        ],
        out_shape=[
            jax.ShapeDtypeStruct((_B * _NR, 1, _RB), jnp.int32),
            jax.ShapeDtypeStruct((_B * _NR, 1, _RB), jnp.int32),
            jax.ShapeDtypeStruct((_B, 1, _N), jnp.float32),
            jax.ShapeDtypeStruct((_B * _NR, 1, _RB), jnp.float32),
            jax.ShapeDtypeStruct((1,), jnp.float32),
        ],
        scratch_shapes=[pltpu.SMEM((1,), jnp.float32)],
        compiler_params=pltpu.CompilerParams(
            dimension_semantics=("arbitrary", "arbitrary")),
    )(src_i, tgt_i, pred, gt, att)
    return out


def _sc_body(pred2d, cg_hbm, ca_hbm, cgt_hbm, p1_hbm, src_hbm, out_hbm,
             ca_v, p1_v, cgt_v, cg_v, src_v, idx_v, col_v, msk_v,
             rows0_v, rows1_v, acc_v, sem):
    wid = lax.axis_index("s") * 2 + lax.axis_index("c")
    base = wid * _CHUNK
    b = base // _N
    ibase = base - b * _N
    pltpu.sync_copy(ca_hbm.at[pl.ds(base, _CHUNK)], ca_v)
    pltpu.sync_copy(p1_hbm.at[pl.ds(base, _CHUNK)], p1_v)
    pltpu.sync_copy(cgt_hbm.at[pl.ds(b * _N, _N)], cgt_v)
    pltpu.sync_copy(cg_hbm.at[pl.ds(b * _N, _N)], cg_v)
    pltpu.sync_copy(src_hbm, src_v)
    s_vec = plsc.load_gather(src_v, [jnp.full((16,), b, jnp.int32)])
    lanes = lax.iota(jnp.int32, 16)
    rowbase = b * ((_N * _N) // 128)
    for g in range(_GROUPS):
        i16 = ibase + g * 16 + lanes
        a16 = ca_v[pl.ds(g * 16, 16)]
        m16 = plsc.load_gather(cgt_v, [a16]).astype(jnp.int32)
        sc16 = plsc.load_gather(cg_v, [m16])
        ridx = rowbase + i16 * (_N // 128) + lax.shift_right_logical(sc16, 7)
        idx_v[g // 8, pl.ds((g % 8) * 16, 16)] = ridx
        col_v[pl.ds(g * 16, 16)] = lax.bitwise_and(sc16, 127)
        mask = (i16 < s_vec) & (m16 != i16)
        msk_v[pl.ds(g * 16, 16)] = mask.astype(jnp.int32)
    d0 = pltpu.async_copy(pred2d.at[idx_v.at[0]], rows0_v, sem)
    d1 = pltpu.async_copy(pred2d.at[idx_v.at[1]], rows1_v, sem)
    d0.wait()
    d1.wait()
    acc = jnp.zeros((16,), jnp.float32)
    for g in range(_GROUPS):
        loc = (g % 8) * 16 + lanes
        col16 = col_v[pl.ds(g * 16, 16)]
        if g < 8:
            p2 = plsc.load_gather(rows0_v, [loc, col16])
        else:
            p2 = plsc.load_gather(rows1_v, [loc, col16])
        p1g = p1_v[pl.ds(g * 16, 16)]
        mask = msk_v[pl.ds(g * 16, 16)] != 0
        acc = acc + jnp.where(mask, p1g - p2, jnp.zeros((16,), jnp.float32))
    acc_v[...] = acc
    pltpu.sync_copy(acc_v, out_hbm.at[wid])


def _make_sc_kernel():
    return functools.partial(
        pl.kernel,
        mesh=plsc.VectorSubcoreMesh(core_axis_name="c", subcore_axis_name="s"),
        out_type=jax.ShapeDtypeStruct((_NW, 16), jnp.float32),
        compiler_params=pltpu.CompilerParams(needs_layout_passes=False),
        scratch_types=[
            pltpu.VMEM((_CHUNK,), jnp.int32),
            pltpu.VMEM((_CHUNK,), jnp.float32),
            pltpu.VMEM((_N,), jnp.float32),
            pltpu.VMEM((_N,), jnp.int32),
            pltpu.VMEM((16,), jnp.int32),
            pltpu.VMEM((2, 128), jnp.int32),
            pltpu.VMEM((_CHUNK,), jnp.int32),
            pltpu.VMEM((_CHUNK,), jnp.int32),
            pltpu.VMEM((128, 128), jnp.float32),
            pltpu.VMEM((128, 128), jnp.float32),
            pltpu.VMEM((16,), jnp.float32),
            pltpu.SemaphoreType.DMA,
        ],
    )(_sc_body)


def kernel(pred_dsmat, pred_perm, pred_perm_att, gt_perm, src_ns, tgt_ns):
    pred = pred_dsmat.astype(jnp.float32)
    gt = gt_perm.astype(jnp.float32)
    att = pred_perm_att.astype(jnp.float32)
    src_i = src_ns.astype(jnp.int32)
    tgt_i = tgt_ns.astype(jnp.int32)
    cg, ca, cgt, p1, s1 = _tc_pass(pred, gt, att, src_i, tgt_i)
    pred2d = pred.reshape(_B * _N * _N // 128, 128)
    src_pad = jnp.zeros((16,), jnp.int32).at[:_B].set(src_i)
    reg_parts = _make_sc_kernel()(
        pred2d,
        cg.reshape(_B * _N),
        ca.reshape(_B * _N),
        cgt.reshape(_B * _N),
        p1.reshape(_B * _N),
        src_pad,
    )
    total = s1[0] - _REG_RATIO * jnp.sum(reg_parts)
    nsum = jnp.sum(src_i.astype(jnp.float32))
    return total / nsum


# speedup vs baseline: 2.1260x; 1.0059x over previous
import functools

import jax
import jax.numpy as jnp
from jax import lax
from jax.experimental import pallas as pl
from jax.experimental.pallas import tpu as pltpu
from jax.experimental.pallas import tpu_sc as plsc

_B = 4
_N = 2048
_REG_RATIO = 0.1
_RB = 1024
_NR = _N // _RB
_NW = 32
_CHUNK = _B * _N // _NW
_GROUPS = _CHUNK // 16


def _tc_body(src_ref, tgt_ref, pred_ref, gt_ref, att_ref,
             cg_ref, ca_ref, cgt_ref, p1_ref, s1_ref, s1_acc):
    b = pl.program_id(0)
    ir = pl.program_id(1)
    s = src_ref[b]
    t = tgt_ref[b]
    P = pred_ref[0]
    G = gt_ref[0]
    A = att_ref[0]
    rows = lax.broadcasted_iota(jnp.int32, (_RB, _N), 0) + ir * _RB
    cols = lax.broadcasted_iota(jnp.int32, (_RB, _N), 1)
    rv = rows < s
    region = rv & (cols < t)
    l1mp = jnp.maximum(jnp.log(1.0 - P), -100.0)
    s1 = jnp.sum(jnp.where(region, -l1mp, 0.0))
    colf = cols.astype(jnp.float32)
    rowf = rows.astype(jnp.float32)
    cg = jnp.sum(G * colf, axis=1)
    ca = jnp.sum(A * colf, axis=1)
    p1 = jnp.sum(P * G, axis=1)
    lp1 = jnp.maximum(jnp.log(p1), -100.0)
    l1mp1 = jnp.maximum(jnp.log(1.0 - p1), -100.0)
    corr = jnp.sum(jnp.where(rv[:, 0], l1mp1 - lp1, 0.0))
    cg_ref[0, 0, :] = cg.astype(jnp.int32)
    ca_ref[0, 0, :] = ca.astype(jnp.int32)
    p1_ref[0, 0, :] = p1
    cgt_part = jnp.sum(G * rowf, axis=0)

    @pl.when(ir == 0)
    def _init():
        cgt_ref[0, 0, :] = cgt_part

    @pl.when(ir != 0)
    def _acc():
        cgt_ref[0, 0, :] = cgt_ref[0, 0, :] + cgt_part

    part = s1 + corr

    @pl.when((b == 0) & (ir == 0))
    def _first():
        s1_acc[0] = part

    @pl.when((b != 0) | (ir != 0))
    def _rest():
        s1_acc[0] = s1_acc[0] + part

    @pl.when((b == _B - 1) & (ir == _NR - 1))
    def _flush():
        s1_ref[0] = s1_acc[0]


def _tc_pass(pred, gt, att, src_i, tgt_i):
    blk = lambda b, ir: (b, ir, 0)
    out = pl.pallas_call(
        _tc_body,
        grid=(_B, _NR),
        in_specs=[
            pl.BlockSpec(memory_space=pltpu.SMEM),
            pl.BlockSpec(memory_space=pltpu.SMEM),
            pl.BlockSpec((1, _RB, _N), blk),
            pl.BlockSpec((1, _RB, _N), blk),
            pl.BlockSpec((1, _RB, _N), blk),
        ],
        out_specs=[
            pl.BlockSpec((1, 1, _RB), lambda b, ir: (b * _NR + ir, 0, 0)),
            pl.BlockSpec((1, 1, _RB), lambda b, ir: (b * _NR + ir, 0, 0)),
            pl.BlockSpec((1, 1, _N), lambda b, ir: (b, 0, 0)),
            pl.BlockSpec((1, 1, _RB), lambda b, ir: (b * _NR + ir, 0, 0)),
            pl.BlockSpec(memory_space=pltpu.SMEM),
        ],
        out_shape=[
            jax.ShapeDtypeStruct((_B * _NR, 1, _RB), jnp.int32),
            jax.ShapeDtypeStruct((_B * _NR, 1, _RB), jnp.int32),
            jax.ShapeDtypeStruct((_B, 1, _N), jnp.float32),
            jax.ShapeDtypeStruct((_B * _NR, 1, _RB), jnp.float32),
            jax.ShapeDtypeStruct((1,), jnp.float32),
        ],
        scratch_shapes=[pltpu.SMEM((1,), jnp.float32)],
        compiler_params=pltpu.CompilerParams(
            dimension_semantics=("arbitrary", "arbitrary")),
    )(src_i, tgt_i, pred, gt, att)
    return out


def _sc_body(pred2d, cg_hbm, ca_hbm, cgt_hbm, p1_hbm, src_hbm, out_hbm,
             ca_v, p1_v, cgt_v, cg_v, src_v, idx_v, col_v, msk_v,
             rows0_v, rows1_v, acc_v, sem):
    wid = lax.axis_index("s") * 2 + lax.axis_index("c")
    base = wid * _CHUNK
    b = base // _N
    ibase = base - b * _N
    pltpu.sync_copy(ca_hbm.at[pl.ds(base, _CHUNK)], ca_v)
    pltpu.sync_copy(p1_hbm.at[pl.ds(base, _CHUNK)], p1_v)
    pltpu.sync_copy(cgt_hbm.at[pl.ds(b * _N, _N)], cgt_v)
    pltpu.sync_copy(cg_hbm.at[pl.ds(b * _N, _N)], cg_v)
    pltpu.sync_copy(src_hbm, src_v)
    s_vec = plsc.load_gather(src_v, [jnp.full((16,), b, jnp.int32)])
    lanes = lax.iota(jnp.int32, 16)
    rowbase = b * ((_N * _N) // 128)
    for g in range(_GROUPS):
        i16 = ibase + g * 16 + lanes
        a16 = ca_v[pl.ds(g * 16, 16)]
        m16 = plsc.load_gather(cgt_v, [a16]).astype(jnp.int32)
        sc16 = plsc.load_gather(cg_v, [m16])
        ridx = rowbase + i16 * (_N // 128) + lax.shift_right_logical(sc16, 7)
        idx_v[g // 8, pl.ds((g % 8) * 16, 16)] = ridx
        col_v[pl.ds(g * 16, 16)] = lax.bitwise_and(sc16, 127)
        mask = (i16 < s_vec) & (m16 != i16)
        msk_v[pl.ds(g * 16, 16)] = mask.astype(jnp.int32)
    d0 = pltpu.async_copy(pred2d.at[idx_v.at[0]], rows0_v, sem)
    d1 = pltpu.async_copy(pred2d.at[idx_v.at[1]], rows1_v, sem)
    d0.wait()
    d1.wait()
    acc = jnp.zeros((16,), jnp.float32)
    for g in range(_GROUPS):
        loc = (g % 8) * 16 + lanes
        col16 = col_v[pl.ds(g * 16, 16)]
        if g < 8:
            p2 = plsc.load_gather(rows0_v, [loc, col16])
        else:
            p2 = plsc.load_gather(rows1_v, [loc, col16])
        p1g = p1_v[pl.ds(g * 16, 16)]
        mask = msk_v[pl.ds(g * 16, 16)] != 0
        acc = acc + jnp.where(mask, p1g - p2, jnp.zeros((16,), jnp.float32))
    acc_v[...] = acc
    pltpu.sync_copy(acc_v, out_hbm.at[wid])


def _make_sc_kernel():
    return functools.partial(
        pl.kernel,
        mesh=plsc.VectorSubcoreMesh(core_axis_name="c", subcore_axis_name="s"),
        out_type=jax.ShapeDtypeStruct((_NW, 16), jnp.float32),
        compiler_params=pltpu.CompilerParams(needs_layout_passes=False),
        scratch_types=[
            pltpu.VMEM((_CHUNK,), jnp.int32),
            pltpu.VMEM((_CHUNK,), jnp.float32),
            pltpu.VMEM((_N,), jnp.float32),
            pltpu.VMEM((_N,), jnp.int32),
            pltpu.VMEM((16,), jnp.int32),
            pltpu.VMEM((2, 128), jnp.int32),
            pltpu.VMEM((_CHUNK,), jnp.int32),
            pltpu.VMEM((_CHUNK,), jnp.int32),
            pltpu.VMEM((128, 128), jnp.float32),
            pltpu.VMEM((128, 128), jnp.float32),
            pltpu.VMEM((16,), jnp.float32),
            pltpu.SemaphoreType.DMA,
        ],
    )(_sc_body)


def kernel(pred_dsmat, pred_perm, pred_perm_att, gt_perm, src_ns, tgt_ns):
    pred = pred_dsmat.astype(jnp.float32)
    gt = gt_perm.astype(jnp.float32)
    att = pred_perm_att.astype(jnp.float32)
    src_i = src_ns.astype(jnp.int32)
    tgt_i = tgt_ns.astype(jnp.int32)
    cg, ca, cgt, p1, s1 = _tc_pass(pred, gt, att, src_i, tgt_i)
    pred2d = pred.reshape(_B * _N * _N // 128, 128)
    src_pad = jnp.zeros((16,), jnp.int32).at[:_B].set(src_i)
    reg_parts = _make_sc_kernel()(
        pred2d,
        cg.reshape(_B * _N),
        ca.reshape(_B * _N),
        cgt.reshape(_B * _N),
        p1.reshape(_B * _N),
        src_pad,
    )
    total = s1[0] - _REG_RATIO * jnp.sum(reg_parts)
    nsum = jnp.sum(src_i.astype(jnp.float32))
    return total / nsum


# speedup vs baseline: 2.2858x; 1.0752x over previous
import functools

import jax
import jax.numpy as jnp
from jax import lax
from jax.experimental import pallas as pl
from jax.experimental.pallas import tpu as pltpu
from jax.experimental.pallas import tpu_sc as plsc

_B = 4
_N = 2048
_REG_RATIO = 0.1
_RB = 1024
_NR = _N // _RB
_NW = 32
_CHUNK = _B * _N // _NW
_GROUPS = _CHUNK // 16
_RCH = 8
_NCH = _CHUNK // _RCH


def _tc_body(src_ref, tgt_ref, pred_ref, gt_ref,
             cg_ref, cgt_ref, p1_ref, s1_ref, s1_acc):
    b = pl.program_id(0)
    ir = pl.program_id(1)
    s = src_ref[b]
    t = tgt_ref[b]
    P = pred_ref[0]
    G = gt_ref[0]
    rows = lax.broadcasted_iota(jnp.int32, (_RB, _N), 0) + ir * _RB
    cols = lax.broadcasted_iota(jnp.int32, (_RB, _N), 1)
    rv = rows < s
    region = rv & (cols < t)
    l1mp = jnp.maximum(jnp.log(1.0 - P), -100.0)
    s1 = jnp.sum(jnp.where(region, -l1mp, 0.0))
    colf = cols.astype(jnp.float32)
    rowf = rows.astype(jnp.float32)
    cg = jnp.sum(G * colf, axis=1)
    p1 = jnp.sum(P * G, axis=1)
    lp1 = jnp.maximum(jnp.log(p1), -100.0)
    l1mp1 = jnp.maximum(jnp.log(1.0 - p1), -100.0)
    corr = jnp.sum(jnp.where(rv[:, 0], l1mp1 - lp1, 0.0))
    cg_ref[0, 0, :] = cg.astype(jnp.int32)
    p1_ref[0, 0, :] = p1
    cgt_part = jnp.sum(G * rowf, axis=0)

    @pl.when(ir == 0)
    def _init():
        cgt_ref[0, 0, :] = cgt_part

    @pl.when(ir != 0)
    def _acc():
        cgt_ref[0, 0, :] = cgt_ref[0, 0, :] + cgt_part

    part = s1 + corr

    @pl.when((b == 0) & (ir == 0))
    def _first():
        s1_acc[0] = part

    @pl.when((b != 0) | (ir != 0))
    def _rest():
        s1_acc[0] = s1_acc[0] + part

    @pl.when((b == _B - 1) & (ir == _NR - 1))
    def _flush():
        s1_ref[0] = s1_acc[0]


def _tc_pass(pred, gt, src_i, tgt_i):
    blk = lambda b, ir: (b, ir, 0)
    return pl.pallas_call(
        _tc_body,
        grid=(_B, _NR),
        in_specs=[
            pl.BlockSpec(memory_space=pltpu.SMEM),
            pl.BlockSpec(memory_space=pltpu.SMEM),
            pl.BlockSpec((1, _RB, _N), blk),
            pl.BlockSpec((1, _RB, _N), blk),
        ],
        out_specs=[
            pl.BlockSpec((1, 1, _RB), lambda b, ir: (b * _NR + ir, 0, 0)),
            pl.BlockSpec((1, 1, _N), lambda b, ir: (b, 0, 0)),
            pl.BlockSpec((1, 1, _RB), lambda b, ir: (b * _NR + ir, 0, 0)),
            pl.BlockSpec(memory_space=pltpu.SMEM),
        ],
        out_shape=[
            jax.ShapeDtypeStruct((_B * _NR, 1, _RB), jnp.int32),
            jax.ShapeDtypeStruct((_B, 1, _N), jnp.float32),
            jax.ShapeDtypeStruct((_B * _NR, 1, _RB), jnp.float32),
            jax.ShapeDtypeStruct((1,), jnp.float32),
        ],
        scratch_shapes=[pltpu.SMEM((1,), jnp.float32)],
        compiler_params=pltpu.CompilerParams(
            dimension_semantics=("arbitrary", "arbitrary")),
    )(src_i, tgt_i, pred, gt)


def _att_body(att_hbm, out_hbm, va0, va1, ca_v, sem0, sem1):
    wid = lax.axis_index("s") * 2 + lax.axis_index("c")
    base = wid * _CHUNK
    lanes = lax.iota(jnp.int32, 16)
    lanesf = lanes.astype(jnp.float32)
    pltpu.async_copy(att_hbm.at[pl.ds(base, _RCH)], va0, sem0)

    def row_sum(va, k):
        def jbody(j, acc):
            o = j * 64
            v0 = va[k, pl.ds(o, 16)]
            v1 = va[k, pl.ds(o + 16, 16)]
            v2 = va[k, pl.ds(o + 32, 16)]
            v3 = va[k, pl.ds(o + 48, 16)]
            jf = o.astype(jnp.float32) + lanesf
            return (acc + v0 * jf + v1 * (jf + 16.0)
                    + v2 * (jf + 32.0) + v3 * (jf + 48.0))
        acc = lax.fori_loop(0, _N // 64, jbody, jnp.zeros((16,), jnp.float32))
        return jnp.sum(acc)

    def chunk_body(c, ca_vec):
        even = (c & 1) == 0
        nxt = c + 1

        @pl.when(even)
        def _w0():
            pltpu.make_async_copy(att_hbm.at[pl.ds(0, _RCH)], va0, sem0).wait()

        @pl.when(~even)
        def _w1():
            pltpu.make_async_copy(att_hbm.at[pl.ds(0, _RCH)], va1, sem1).wait()

        @pl.when(nxt < _NCH)
        def _issue():
            off = base + nxt * _RCH

            @pl.when(even)
            def _i1():
                pltpu.async_copy(att_hbm.at[pl.ds(off, _RCH)], va1, sem1)

            @pl.when(~even)
            def _i0():
                pltpu.async_copy(att_hbm.at[pl.ds(off, _RCH)], va0, sem0)

        half = (c & 1) * 8
        for k in range(_RCH):
            s0 = lax.cond(even,
                          lambda: row_sum(va0, k),
                          lambda: row_sum(va1, k))
            ca_vec = jnp.where(lanes == half + k, s0, ca_vec)

        @pl.when(c & 1 == 1)
        def _store():
            ca_v[pl.ds((c // 2) * 16, 16)] = ca_vec.astype(jnp.int32)

        return ca_vec

    lax.fori_loop(0, _NCH, chunk_body, jnp.zeros((16,), jnp.float32))
    pltpu.sync_copy(ca_v, out_hbm.at[pl.ds(base, _CHUNK)])


def _make_att_kernel():
    return functools.partial(
        pl.kernel,
        mesh=plsc.VectorSubcoreMesh(core_axis_name="c", subcore_axis_name="s"),
        out_type=jax.ShapeDtypeStruct((_B * _N,), jnp.int32),
        compiler_params=pltpu.CompilerParams(needs_layout_passes=False),
        scratch_types=[
            pltpu.VMEM((_RCH, _N), jnp.float32),
            pltpu.VMEM((_RCH, _N), jnp.float32),
            pltpu.VMEM((_CHUNK,), jnp.int32),
            pltpu.SemaphoreType.DMA,
            pltpu.SemaphoreType.DMA,
        ],
    )(_att_body)


def _sc_body(pred2d, cg_hbm, ca_hbm, cgt_hbm, p1_hbm, src_hbm, out_hbm,
             ca_v, p1_v, cgt_v, cg_v, src_v, idx_v, col_v, msk_v,
             rows0_v, rows1_v, acc_v, sem):
    wid = lax.axis_index("s") * 2 + lax.axis_index("c")
    base = wid * _CHUNK
    b = base // _N
    ibase = base - b * _N
    pltpu.sync_copy(ca_hbm.at[pl.ds(base, _CHUNK)], ca_v)
    pltpu.sync_copy(p1_hbm.at[pl.ds(base, _CHUNK)], p1_v)
    pltpu.sync_copy(cgt_hbm.at[pl.ds(b * _N, _N)], cgt_v)
    pltpu.sync_copy(cg_hbm.at[pl.ds(b * _N, _N)], cg_v)
    pltpu.sync_copy(src_hbm, src_v)
    s_vec = plsc.load_gather(src_v, [jnp.full((16,), b, jnp.int32)])
    lanes = lax.iota(jnp.int32, 16)
    rowbase = b * ((_N * _N) // 128)
    for g in range(_GROUPS):
        i16 = ibase + g * 16 + lanes
        a16 = ca_v[pl.ds(g * 16, 16)]
        m16 = plsc.load_gather(cgt_v, [a16]).astype(jnp.int32)
        sc16 = plsc.load_gather(cg_v, [m16])
        ridx = rowbase + i16 * (_N // 128) + lax.shift_right_logical(sc16, 7)
        idx_v[g // 8, pl.ds((g % 8) * 16, 16)] = ridx
        col_v[pl.ds(g * 16, 16)] = lax.bitwise_and(sc16, 127)
        mask = (i16 < s_vec) & (m16 != i16)
        msk_v[pl.ds(g * 16, 16)] = mask.astype(jnp.int32)
    d0 = pltpu.async_copy(pred2d.at[idx_v.at[0]], rows0_v, sem)
    d1 = pltpu.async_copy(pred2d.at[idx_v.at[1]], rows1_v, sem)
    d0.wait()
    d1.wait()
    acc = jnp.zeros((16,), jnp.float32)
    for g in range(_GROUPS):
        loc = (g % 8) * 16 + lanes
        col16 = col_v[pl.ds(g * 16, 16)]
        if g < 8:
            p2 = plsc.load_gather(rows0_v, [loc, col16])
        else:
            p2 = plsc.load_gather(rows1_v, [loc, col16])
        p1g = p1_v[pl.ds(g * 16, 16)]
        mask = msk_v[pl.ds(g * 16, 16)] != 0
        acc = acc + jnp.where(mask, p1g - p2, jnp.zeros((16,), jnp.float32))
    acc_v[...] = acc
    pltpu.sync_copy(acc_v, out_hbm.at[wid])


def _make_sc_kernel():
    return functools.partial(
        pl.kernel,
        mesh=plsc.VectorSubcoreMesh(core_axis_name="c", subcore_axis_name="s"),
        out_type=jax.ShapeDtypeStruct((_NW, 16), jnp.float32),
        compiler_params=pltpu.CompilerParams(needs_layout_passes=False),
        scratch_types=[
            pltpu.VMEM((_CHUNK,), jnp.int32),
            pltpu.VMEM((_CHUNK,), jnp.float32),
            pltpu.VMEM((_N,), jnp.float32),
            pltpu.VMEM((_N,), jnp.int32),
            pltpu.VMEM((16,), jnp.int32),
            pltpu.VMEM((2, 128), jnp.int32),
            pltpu.VMEM((_CHUNK,), jnp.int32),
            pltpu.VMEM((_CHUNK,), jnp.int32),
            pltpu.VMEM((128, 128), jnp.float32),
            pltpu.VMEM((128, 128), jnp.float32),
            pltpu.VMEM((16,), jnp.float32),
            pltpu.SemaphoreType.DMA,
        ],
    )(_sc_body)


def kernel(pred_dsmat, pred_perm, pred_perm_att, gt_perm, src_ns, tgt_ns):
    pred = pred_dsmat.astype(jnp.float32)
    gt = gt_perm.astype(jnp.float32)
    att = pred_perm_att.astype(jnp.float32)
    src_i = src_ns.astype(jnp.int32)
    tgt_i = tgt_ns.astype(jnp.int32)
    ca = _make_att_kernel()(att.reshape(_B * _N, _N))
    cg, cgt, p1, s1 = _tc_pass(pred, gt, src_i, tgt_i)
    pred2d = pred.reshape(_B * _N * _N // 128, 128)
    src_pad = jnp.zeros((16,), jnp.int32).at[:_B].set(src_i)
    reg_parts = _make_sc_kernel()(
        pred2d,
        cg.reshape(_B * _N),
        ca,
        cgt.reshape(_B * _N),
        p1.reshape(_B * _N),
        src_pad,
    )
    total = s1[0] - _REG_RATIO * jnp.sum(reg_parts)
    nsum = jnp.sum(src_i.astype(jnp.float32))
    return total / nsum


# speedup vs baseline: 2.8652x; 1.2535x over previous
import functools

import jax
import jax.numpy as jnp
from jax import lax
from jax.experimental import pallas as pl
from jax.experimental.pallas import tpu as pltpu
from jax.experimental.pallas import tpu_sc as plsc

_B = 4
_N = 2048
_REG_RATIO = 0.1
_RB = 1024
_NR = _N // _RB
_NW = 32
_CHUNK = _B * _N // _NW
_GROUPS = _CHUNK // 16
_RCH = 8
_NCH = _CHUNK // _RCH


def _att_body(att_hbm, out_hbm, va0, va1, ca_v, sem0, sem1):
    wid = lax.axis_index("s") * 2 + lax.axis_index("c")
    base = wid * _CHUNK
    lanes = lax.iota(jnp.int32, 16)
    lanesf = lanes.astype(jnp.float32)
    pltpu.async_copy(att_hbm.at[pl.ds(base, _RCH)], va0, sem0)

    def row_sum(va, k):
        def jbody(j, acc):
            o = j * 64
            v0 = va[k, pl.ds(o, 16)]
            v1 = va[k, pl.ds(o + 16, 16)]
            v2 = va[k, pl.ds(o + 32, 16)]
            v3 = va[k, pl.ds(o + 48, 16)]
            jf = o.astype(jnp.float32) + lanesf
            return (acc + v0 * jf + v1 * (jf + 16.0)
                    + v2 * (jf + 32.0) + v3 * (jf + 48.0))
        acc = lax.fori_loop(0, _N // 64, jbody, jnp.zeros((16,), jnp.float32))
        return jnp.sum(acc)

    def chunk_body(c, ca_vec):
        even = (c & 1) == 0
        nxt = c + 1

        @pl.when(even)
        def _w0():
            pltpu.make_async_copy(att_hbm.at[pl.ds(0, _RCH)], va0, sem0).wait()

        @pl.when(~even)
        def _w1():
            pltpu.make_async_copy(att_hbm.at[pl.ds(0, _RCH)], va1, sem1).wait()

        @pl.when(nxt < _NCH)
        def _issue():
            off = base + nxt * _RCH

            @pl.when(even)
            def _i1():
                pltpu.async_copy(att_hbm.at[pl.ds(off, _RCH)], va1, sem1)

            @pl.when(~even)
            def _i0():
                pltpu.async_copy(att_hbm.at[pl.ds(off, _RCH)], va0, sem0)

        half = (c & 1) * 8
        for k in range(_RCH):
            s0 = lax.cond(even,
                          lambda: row_sum(va0, k),
                          lambda: row_sum(va1, k))
            ca_vec = jnp.where(lanes == half + k, s0, ca_vec)

        @pl.when((c & 1) == 1)
        def _store():
            ca_v[pl.ds((c // 2) * 16, 16)] = ca_vec.astype(jnp.int32)

        return ca_vec

    lax.fori_loop(0, _NCH, chunk_body, jnp.zeros((16,), jnp.float32))
    pltpu.sync_copy(ca_v, out_hbm.at[pl.ds(base, _CHUNK)])


def _make_att_kernel():
    return functools.partial(
        pl.kernel,
        mesh=plsc.VectorSubcoreMesh(core_axis_name="c", subcore_axis_name="s"),
        out_type=jax.ShapeDtypeStruct((_B * _N,), jnp.int32),
        compiler_params=pltpu.CompilerParams(needs_layout_passes=False),
        scratch_types=[
            pltpu.VMEM((_RCH, _N), jnp.float32),
            pltpu.VMEM((_RCH, _N), jnp.float32),
            pltpu.VMEM((_CHUNK,), jnp.int32),
            pltpu.SemaphoreType.DMA,
            pltpu.SemaphoreType.DMA,
        ],
    )(_att_body)


def _gt_body(gt_ref, cg_ref, cgt_ref):
    ir = pl.program_id(1)
    G = gt_ref[0]
    rows = lax.broadcasted_iota(jnp.int32, (_RB, _N), 0) + ir * _RB
    cols = lax.broadcasted_iota(jnp.int32, (_RB, _N), 1)
    cg_ref[0, 0, :] = jnp.sum(G * cols.astype(jnp.float32), axis=1).astype(jnp.int32)
    cgt_part = jnp.sum(G * rows.astype(jnp.float32), axis=0)

    @pl.when(ir == 0)
    def _init():
        cgt_ref[0, 0, :] = cgt_part

    @pl.when(ir != 0)
    def _acc():
        cgt_ref[0, 0, :] = cgt_ref[0, 0, :] + cgt_part


def _gt_pass(gt):
    return pl.pallas_call(
        _gt_body,
        grid=(_B, _NR),
        in_specs=[pl.BlockSpec((1, _RB, _N), lambda b, ir: (b, ir, 0))],
        out_specs=[
            pl.BlockSpec((1, 1, _RB), lambda b, ir: (b * _NR + ir, 0, 0)),
            pl.BlockSpec((1, 1, _N), lambda b, ir: (b, 0, 0)),
        ],
        out_shape=[
            jax.ShapeDtypeStruct((_B * _NR, 1, _RB), jnp.int32),
            jax.ShapeDtypeStruct((_B, 1, _N), jnp.float32),
        ],
        compiler_params=pltpu.CompilerParams(
            dimension_semantics=("arbitrary", "arbitrary")),
    )(gt)


def _chase_body(cg_hbm, ca_hbm, cgt_hbm, src_hbm, sc_out, fl_out,
                ca_v, cgt_v, cg_v, src_v, scv, flv, sem):
    del sem
    wid = lax.axis_index("s") * 2 + lax.axis_index("c")
    base = wid * _CHUNK
    b = base // _N
    ibase = base - b * _N
    pltpu.sync_copy(ca_hbm.at[pl.ds(base, _CHUNK)], ca_v)
    pltpu.sync_copy(cgt_hbm.at[pl.ds(b * _N, _N)], cgt_v)
    pltpu.sync_copy(cg_hbm.at[pl.ds(b * _N, _N)], cg_v)
    pltpu.sync_copy(src_hbm, src_v)
    s_vec = plsc.load_gather(src_v, [jnp.full((16,), b, jnp.int32)])
    lanes = lax.iota(jnp.int32, 16)
    for g in range(_GROUPS):
        i16 = ibase + g * 16 + lanes
        a16 = ca_v[pl.ds(g * 16, 16)]
        m16 = plsc.load_gather(cgt_v, [a16]).astype(jnp.int32)
        sc16 = plsc.load_gather(cg_v, [m16])
        flag = (i16 < s_vec) & (m16 != i16)
        scv[pl.ds(g * 16, 16)] = sc16
        flv[pl.ds(g * 16, 16)] = flag.astype(jnp.int32)
    pltpu.sync_copy(scv, sc_out.at[pl.ds(base, _CHUNK)])
    pltpu.sync_copy(flv, fl_out.at[pl.ds(base, _CHUNK)])


def _make_chase_kernel():
    return functools.partial(
        pl.kernel,
        mesh=plsc.VectorSubcoreMesh(core_axis_name="c", subcore_axis_name="s"),
        out_type=[
            jax.ShapeDtypeStruct((_B * _N,), jnp.int32),
            jax.ShapeDtypeStruct((_B * _N,), jnp.int32),
        ],
        compiler_params=pltpu.CompilerParams(needs_layout_passes=False),
        scratch_types=[
            pltpu.VMEM((_CHUNK,), jnp.int32),
            pltpu.VMEM((_N,), jnp.float32),
            pltpu.VMEM((_N,), jnp.int32),
            pltpu.VMEM((16,), jnp.int32),
            pltpu.VMEM((_CHUNK,), jnp.int32),
            pltpu.VMEM((_CHUNK,), jnp.int32),
            pltpu.SemaphoreType.DMA,
        ],
    )(_chase_body)


def _pred_body(src_ref, tgt_ref, pred_ref, cg_ref, sc_ref, fl_ref,
               out_ref, acc_ref):
    b = pl.program_id(0)
    ir = pl.program_id(1)
    s = src_ref[b]
    t = tgt_ref[b]
    P = pred_ref[0]
    rows = lax.broadcasted_iota(jnp.int32, (_RB, _N), 0) + ir * _RB
    cols = lax.broadcasted_iota(jnp.int32, (_RB, _N), 1)
    rv = rows < s
    region = rv & (cols < t)
    l1mp = jnp.maximum(jnp.log(1.0 - P), -100.0)
    s1 = jnp.sum(jnp.where(region, -l1mp, 0.0))
    cgv = cg_ref[0, 0, :][:, None]
    scv = sc_ref[0, 0, :][:, None]
    p1 = jnp.sum(jnp.where(cols == cgv, P, 0.0), axis=1)
    p2 = jnp.sum(jnp.where(cols == scv, P, 0.0), axis=1)
    lp1 = jnp.maximum(jnp.log(p1), -100.0)
    l1mp1 = jnp.maximum(jnp.log(1.0 - p1), -100.0)
    corr = jnp.sum(jnp.where(rv[:, 0], l1mp1 - lp1, 0.0))
    flg = fl_ref[0, 0, :] != 0
    reg = jnp.sum(jnp.where(flg, p1 - p2, 0.0))
    part = s1 + corr - _REG_RATIO * reg

    @pl.when((b == 0) & (ir == 0))
    def _first():
        acc_ref[0] = part

    @pl.when((b != 0) | (ir != 0))
    def _rest():
        acc_ref[0] = acc_ref[0] + part

    @pl.when((b == _B - 1) & (ir == _NR - 1))
    def _flush():
        nsum = (src_ref[0] + src_ref[1] + src_ref[2] + src_ref[3]).astype(jnp.float32)
        out_ref[0] = acc_ref[0] / nsum


def _pred_pass(pred, cg, sc, fl, src_i, tgt_i):
    rowspec = pl.BlockSpec((1, 1, _RB), lambda b, ir: (b * _NR + ir, 0, 0))
    return pl.pallas_call(
        _pred_body,
        grid=(_B, _NR),
        in_specs=[
            pl.BlockSpec(memory_space=pltpu.SMEM),
            pl.BlockSpec(memory_space=pltpu.SMEM),
            pl.BlockSpec((1, _RB, _N), lambda b, ir: (b, ir, 0)),
            rowspec,
            rowspec,
            rowspec,
        ],
        out_specs=[pl.BlockSpec(memory_space=pltpu.SMEM)],
        out_shape=[jax.ShapeDtypeStruct((1,), jnp.float32)],
        scratch_shapes=[pltpu.SMEM((1,), jnp.float32)],
        compiler_params=pltpu.CompilerParams(
            dimension_semantics=("arbitrary", "arbitrary")),
    )(src_i, tgt_i, pred, cg, sc, fl)


def kernel(pred_dsmat, pred_perm, pred_perm_att, gt_perm, src_ns, tgt_ns):
    pred = pred_dsmat.astype(jnp.float32)
    gt = gt_perm.astype(jnp.float32)
    att = pred_perm_att.astype(jnp.float32)
    src_i = src_ns.astype(jnp.int32)
    tgt_i = tgt_ns.astype(jnp.int32)
    ca = _make_att_kernel()(att.reshape(_B * _N, _N))
    cg, cgt = _gt_pass(gt)
    src_pad = jnp.zeros((16,), jnp.int32).at[:_B].set(src_i)
    sc, fl = _make_chase_kernel()(
        cg.reshape(_B * _N), ca, cgt.reshape(_B * _N), src_pad)
    out = _pred_pass(
        pred,
        cg,
        sc.reshape(_B * _NR, 1, _RB),
        fl.reshape(_B * _NR, 1, _RB),
        src_i,
        tgt_i,
    )[0]
    return out[0]
